# Initial kernel scaffold; baseline (speedup 1.0000x reference)
import functools

import jax
import jax.numpy as jnp
from jax import lax
from jax.experimental import pallas as pl
from jax.experimental.pallas import tpu as pltpu
from jax.experimental.pallas import tpu_sc as plsc

BATCH = 16384
EMBED_DIM = 32
CONT_DIM = 64

_NC = 2
_NS = 16
_NW = _NC * _NS
_B_PER_W = BATCH // _NW
_CHUNK = 128
_N_CHUNKS = _B_PER_W // _CHUNK


def _sc_gather_body(flag_tab, class_tab, fidx_hbm, cidx_hbm,
                    fout_hbm, cout_hbm,
                    fidx_v, cidx_v, frows_v, crows_v, fsem, csem):
    wid = lax.axis_index("s") * _NC + lax.axis_index("c")
    base = wid * _B_PER_W

    pltpu.sync_copy(fidx_hbm.at[wid], fidx_v)
    pltpu.sync_copy(cidx_hbm.at[wid], cidx_v)

    copies = []
    for j in range(_N_CHUNKS):
        copies.append(pltpu.async_copy(
            flag_tab.at[fidx_v.at[j]],
            frows_v.at[pl.ds(j * _CHUNK, _CHUNK)], fsem))
        copies.append(pltpu.async_copy(
            class_tab.at[cidx_v.at[j]],
            crows_v.at[pl.ds(j * _CHUNK, _CHUNK)], csem))
    for c in copies:
        c.wait()

    pltpu.sync_copy(frows_v, fout_hbm.at[pl.ds(base, _B_PER_W)])
    pltpu.sync_copy(crows_v, cout_hbm.at[pl.ds(base, _B_PER_W)])


def _sc_gather(flag_table, class_table, fidx, cidx):
    mesh = plsc.VectorSubcoreMesh(core_axis_name="c", subcore_axis_name="s")
    kern = functools.partial(
        pl.kernel,
        mesh=mesh,
        out_type=[
            jax.ShapeDtypeStruct((BATCH, EMBED_DIM), jnp.float32),
            jax.ShapeDtypeStruct((BATCH, EMBED_DIM), jnp.float32),
        ],
        scratch_types=[
            pltpu.VMEM((_N_CHUNKS, _CHUNK), jnp.int32),
            pltpu.VMEM((_N_CHUNKS, _CHUNK), jnp.int32),
            pltpu.VMEM((_B_PER_W, EMBED_DIM), jnp.float32),
            pltpu.VMEM((_B_PER_W, EMBED_DIM), jnp.float32),
            pltpu.SemaphoreType.DMA,
            pltpu.SemaphoreType.DMA,
        ],
    )(_sc_gather_body)
    return kern(flag_table, class_table, fidx, cidx)


def _tc_matmul_body(x_ref, f_ref, c_ref, w_ref, b_ref, o_ref):
    acc = jnp.dot(x_ref[...], w_ref[0:CONT_DIM, :],
                  preferred_element_type=jnp.float32)
    acc += jnp.dot(f_ref[...], w_ref[CONT_DIM:CONT_DIM + EMBED_DIM, :],
                   preferred_element_type=jnp.float32)
    acc += jnp.dot(c_ref[...], w_ref[CONT_DIM + EMBED_DIM:, :],
                   preferred_element_type=jnp.float32)
    o_ref[...] = acc + b_ref[...]


def _tc_matmul(x_cont, flag_emb, class_emb, W, b):
    blk = 2048
    grid = (BATCH // blk,)
    return pl.pallas_call(
        _tc_matmul_body,
        grid=grid,
        in_specs=[
            pl.BlockSpec((blk, CONT_DIM), lambda i: (i, 0)),
            pl.BlockSpec((blk, EMBED_DIM), lambda i: (i, 0)),
            pl.BlockSpec((blk, EMBED_DIM), lambda i: (i, 0)),
            pl.BlockSpec((CONT_DIM + 2 * EMBED_DIM, CONT_DIM),
                         lambda i: (0, 0)),
            pl.BlockSpec((1, CONT_DIM), lambda i: (0, 0)),
        ],
        out_specs=pl.BlockSpec((blk, CONT_DIM), lambda i: (i, 0)),
        out_shape=jax.ShapeDtypeStruct((BATCH, CONT_DIM), jnp.float32),
    )(x_cont, flag_emb, class_emb, W, b.reshape(1, CONT_DIM))


def kernel(x_cont, flag_idx, class_idx, flag_table, class_table, W, b):
    fidx = flag_idx.astype(jnp.int32).reshape(_NW, _N_CHUNKS, _CHUNK)
    cidx = class_idx.astype(jnp.int32).reshape(_NW, _N_CHUNKS, _CHUNK)
    flag_emb, class_emb = _sc_gather(flag_table, class_table, fidx, cidx)
    return _tc_matmul(x_cont, flag_emb, class_emb, W, b)


# speedup vs baseline: 1.3936x; 1.3936x over previous
import functools

import jax
import jax.numpy as jnp
from jax import lax
from jax.experimental import pallas as pl
from jax.experimental.pallas import tpu as pltpu
from jax.experimental.pallas import tpu_sc as plsc

BATCH = 16384
EMBED_DIM = 32
CONT_DIM = 64

_NC = 2
_NS = 16
_NW = _NC * _NS
_B_PER_W = BATCH // _NW
_CHUNK = 128
_N_CHUNKS = _B_PER_W // _CHUNK


def _sc_gather_body(flag_tab, class_tab, fidx_hbm, cidx_hbm,
                    fout_hbm, cout_hbm,
                    fidx_v, cidx_v, frows_v, crows_v, fsem, csem):
    wid = lax.axis_index("s") * _NC + lax.axis_index("c")
    base = wid * _B_PER_W

    pltpu.sync_copy(fidx_hbm.at[wid], fidx_v)
    pltpu.sync_copy(cidx_hbm.at[wid], cidx_v)

    copies = []
    for j in range(_N_CHUNKS):
        copies.append(pltpu.async_copy(
            flag_tab.at[fidx_v.at[j]],
            frows_v.at[pl.ds(j * _CHUNK, _CHUNK)], fsem))
        copies.append(pltpu.async_copy(
            class_tab.at[cidx_v.at[j]],
            crows_v.at[pl.ds(j * _CHUNK, _CHUNK)], csem))
    for c in copies:
        c.wait()

    pltpu.sync_copy(frows_v, fout_hbm.at[pl.ds(base, _B_PER_W)])
    pltpu.sync_copy(crows_v, cout_hbm.at[pl.ds(base, _B_PER_W)])


def _sc_gather(flag_table, class_table, fidx, cidx):
    mesh = plsc.VectorSubcoreMesh(core_axis_name="c", subcore_axis_name="s")
    kern = functools.partial(
        pl.kernel,
        mesh=mesh,
        out_type=[
            jax.ShapeDtypeStruct((BATCH, EMBED_DIM), jnp.float32),
            jax.ShapeDtypeStruct((BATCH, EMBED_DIM), jnp.float32),
        ],
        scratch_types=[
            pltpu.VMEM((_N_CHUNKS, _CHUNK), jnp.int32),
            pltpu.VMEM((_N_CHUNKS, _CHUNK), jnp.int32),
            pltpu.VMEM((_B_PER_W, EMBED_DIM), jnp.float32),
            pltpu.VMEM((_B_PER_W, EMBED_DIM), jnp.float32),
            pltpu.SemaphoreType.DMA,
            pltpu.SemaphoreType.DMA,
        ],
        compiler_params=pltpu.CompilerParams(use_tc_tiling_on_sc=False),
    )(_sc_gather_body)
    return kern(flag_table, class_table, fidx, cidx)


def _tc_matmul_body(x_ref, f_ref, c_ref, w_ref, b_ref, o_ref):
    acc = jnp.dot(x_ref[...], w_ref[0:CONT_DIM, :],
                  preferred_element_type=jnp.float32)
    acc += jnp.dot(f_ref[...], w_ref[CONT_DIM:CONT_DIM + EMBED_DIM, :],
                   preferred_element_type=jnp.float32)
    acc += jnp.dot(c_ref[...], w_ref[CONT_DIM + EMBED_DIM:, :],
                   preferred_element_type=jnp.float32)
    o_ref[...] = acc + b_ref[...]


def _tc_matmul(x_cont, flag_emb, class_emb, W, b):
    blk = 2048
    grid = (BATCH // blk,)
    return pl.pallas_call(
        _tc_matmul_body,
        grid=grid,
        in_specs=[
            pl.BlockSpec((blk, CONT_DIM), lambda i: (i, 0)),
            pl.BlockSpec((blk, EMBED_DIM), lambda i: (i, 0)),
            pl.BlockSpec((blk, EMBED_DIM), lambda i: (i, 0)),
            pl.BlockSpec((CONT_DIM + 2 * EMBED_DIM, CONT_DIM),
                         lambda i: (0, 0)),
            pl.BlockSpec((1, CONT_DIM), lambda i: (0, 0)),
        ],
        out_specs=pl.BlockSpec((blk, CONT_DIM), lambda i: (i, 0)),
        out_shape=jax.ShapeDtypeStruct((BATCH, CONT_DIM), jnp.float32),
    )(x_cont, flag_emb, class_emb, W, b.reshape(1, CONT_DIM))


def kernel(x_cont, flag_idx, class_idx, flag_table, class_table, W, b):
    fidx = flag_idx.astype(jnp.int32).reshape(_NW, _N_CHUNKS, _CHUNK)
    cidx = class_idx.astype(jnp.int32).reshape(_NW, _N_CHUNKS, _CHUNK)
    flag_emb, class_emb = _sc_gather(flag_table, class_table, fidx, cidx)
    return _tc_matmul(x_cont, flag_emb, class_emb, W, b)


# speedup vs baseline: 1.3959x; 1.0017x over previous
import functools

import jax
import jax.numpy as jnp
from jax import lax
from jax.experimental import pallas as pl
from jax.experimental.pallas import tpu as pltpu
from jax.experimental.pallas import tpu_sc as plsc

BATCH = 16384
EMBED_DIM = 32
CONT_DIM = 64

_NC = 2
_NS = 16
_NW = _NC * _NS
_B_PER_W = BATCH // _NW
_CHUNK = 128
_N_CHUNKS = _B_PER_W // _CHUNK


def _sc_gather_body(flag_tab, class_tab, fidx_hbm, cidx_hbm,
                    fout_hbm, cout_hbm,
                    fidx_v, cidx_v, frows_v, crows_v, fsem, csem):
    wid = lax.axis_index("s") * _NC + lax.axis_index("c")
    base = wid * _B_PER_W

    pltpu.sync_copy(fidx_hbm.at[pl.ds(base, _B_PER_W)], fidx_v)
    pltpu.sync_copy(cidx_hbm.at[pl.ds(base, _B_PER_W)], cidx_v)

    copies = []
    for j in range(_N_CHUNKS):
        copies.append(pltpu.async_copy(
            flag_tab.at[fidx_v.at[pl.ds(j * _CHUNK, _CHUNK)]],
            frows_v.at[pl.ds(j * _CHUNK, _CHUNK)], fsem))
        copies.append(pltpu.async_copy(
            class_tab.at[cidx_v.at[pl.ds(j * _CHUNK, _CHUNK)]],
            crows_v.at[pl.ds(j * _CHUNK, _CHUNK)], csem))
    for c in copies:
        c.wait()

    pltpu.sync_copy(frows_v, fout_hbm.at[pl.ds(base, _B_PER_W)])
    pltpu.sync_copy(crows_v, cout_hbm.at[pl.ds(base, _B_PER_W)])


def _sc_gather(flag_table, class_table, fidx, cidx):
    mesh = plsc.VectorSubcoreMesh(core_axis_name="c", subcore_axis_name="s")
    kern = functools.partial(
        pl.kernel,
        mesh=mesh,
        out_type=[
            jax.ShapeDtypeStruct((BATCH, EMBED_DIM), jnp.float32),
            jax.ShapeDtypeStruct((BATCH, EMBED_DIM), jnp.float32),
        ],
        scratch_types=[
            pltpu.VMEM((_B_PER_W,), jnp.int32),
            pltpu.VMEM((_B_PER_W,), jnp.int32),
            pltpu.VMEM((_B_PER_W, EMBED_DIM), jnp.float32),
            pltpu.VMEM((_B_PER_W, EMBED_DIM), jnp.float32),
            pltpu.SemaphoreType.DMA,
            pltpu.SemaphoreType.DMA,
        ],
        compiler_params=pltpu.CompilerParams(use_tc_tiling_on_sc=False),
    )(_sc_gather_body)
    return kern(flag_table, class_table, fidx, cidx)


def _tc_matmul_body(x_ref, f_ref, c_ref, w_ref, b_ref, o_ref):
    acc = jnp.dot(x_ref[...], w_ref[0:CONT_DIM, :],
                  preferred_element_type=jnp.float32)
    acc += jnp.dot(f_ref[...], w_ref[CONT_DIM:CONT_DIM + EMBED_DIM, :],
                   preferred_element_type=jnp.float32)
    acc += jnp.dot(c_ref[...], w_ref[CONT_DIM + EMBED_DIM:, :],
                   preferred_element_type=jnp.float32)
    o_ref[...] = acc + b_ref[...]


def _tc_matmul(x_cont, flag_emb, class_emb, W, b):
    blk = 2048
    grid = (BATCH // blk,)
    return pl.pallas_call(
        _tc_matmul_body,
        grid=grid,
        in_specs=[
            pl.BlockSpec((blk, CONT_DIM), lambda i: (i, 0)),
            pl.BlockSpec((blk, EMBED_DIM), lambda i: (i, 0)),
            pl.BlockSpec((blk, EMBED_DIM), lambda i: (i, 0)),
            pl.BlockSpec((CONT_DIM + 2 * EMBED_DIM, CONT_DIM),
                         lambda i: (0, 0)),
            pl.BlockSpec((1, CONT_DIM), lambda i: (0, 0)),
        ],
        out_specs=pl.BlockSpec((blk, CONT_DIM), lambda i: (i, 0)),
        out_shape=jax.ShapeDtypeStruct((BATCH, CONT_DIM), jnp.float32),
    )(x_cont, flag_emb, class_emb, W, b.reshape(1, CONT_DIM))


def kernel(x_cont, flag_idx, class_idx, flag_table, class_table, W, b):
    fidx = flag_idx.astype(jnp.int32)
    cidx = class_idx.astype(jnp.int32)
    flag_emb, class_emb = _sc_gather(flag_table, class_table, fidx, cidx)
    return _tc_matmul(x_cont, flag_emb, class_emb, W, b)


# speedup vs baseline: 1.4449x; 1.0351x over previous
import functools

import jax
import jax.numpy as jnp
from jax import lax
from jax.experimental import pallas as pl
from jax.experimental.pallas import tpu as pltpu
from jax.experimental.pallas import tpu_sc as plsc

BATCH = 16384
EMBED_DIM = 32
PAD_DIM = 128
CONT_DIM = 64

_NC = 2
_NS = 16
_NW = _NC * _NS
_B_PER_W = BATCH // _NW
_CHUNK = 128
_N_CHUNKS = _B_PER_W // _CHUNK


def _sc_gather_body(flag_tab, class_tab, fidx_hbm, cidx_hbm,
                    fout_hbm, cout_hbm,
                    fidx_v, cidx_v, frows_v, crows_v, fsem, csem):
    wid = lax.axis_index("s") * _NC + lax.axis_index("c")
    base = wid * _B_PER_W

    pltpu.sync_copy(fidx_hbm.at[pl.ds(base, _B_PER_W)], fidx_v)
    pltpu.sync_copy(cidx_hbm.at[pl.ds(base, _B_PER_W)], cidx_v)

    copies = []
    for j in range(_N_CHUNKS):
        copies.append(pltpu.async_copy(
            flag_tab.at[fidx_v.at[pl.ds(j * _CHUNK, _CHUNK)]],
            frows_v.at[pl.ds(j * _CHUNK, _CHUNK)], fsem))
        copies.append(pltpu.async_copy(
            class_tab.at[cidx_v.at[pl.ds(j * _CHUNK, _CHUNK)]],
            crows_v.at[pl.ds(j * _CHUNK, _CHUNK)], csem))
    for c in copies:
        c.wait()

    pltpu.sync_copy(frows_v, fout_hbm.at[pl.ds(base, _B_PER_W)])
    pltpu.sync_copy(crows_v, cout_hbm.at[pl.ds(base, _B_PER_W)])


def _sc_gather(flag_table_pad, class_table, fidx, cidx):
    mesh = plsc.VectorSubcoreMesh(core_axis_name="c", subcore_axis_name="s")
    kern = functools.partial(
        pl.kernel,
        mesh=mesh,
        out_type=[
            jax.ShapeDtypeStruct((BATCH, PAD_DIM), jnp.float32),
            jax.ShapeDtypeStruct((BATCH, EMBED_DIM), jnp.float32),
        ],
        scratch_types=[
            pltpu.VMEM((_B_PER_W,), jnp.int32),
            pltpu.VMEM((_B_PER_W,), jnp.int32),
            pltpu.VMEM((_B_PER_W, PAD_DIM), jnp.float32),
            pltpu.VMEM((_B_PER_W, EMBED_DIM), jnp.float32),
            pltpu.SemaphoreType.DMA,
            pltpu.SemaphoreType.DMA,
        ],
        compiler_params=pltpu.CompilerParams(use_tc_tiling_on_sc=False),
    )(_sc_gather_body)
    return kern(flag_table_pad, class_table, fidx, cidx)


def _tc_matmul_body(x_ref, f_ref, c_ref, w_ref, wf_ref, b_ref, o_ref):
    acc = jnp.dot(x_ref[...], w_ref[0:CONT_DIM, :],
                  preferred_element_type=jnp.float32)
    acc += jnp.dot(f_ref[...], wf_ref[...],
                   preferred_element_type=jnp.float32)
    acc += jnp.dot(c_ref[...], w_ref[CONT_DIM + EMBED_DIM:, :],
                   preferred_element_type=jnp.float32)
    o_ref[...] = acc + b_ref[...]


def _tc_matmul(x_cont, flag_emb_pad, class_emb, W, W1pad, b):
    blk = 2048
    grid = (BATCH // blk,)
    return pl.pallas_call(
        _tc_matmul_body,
        grid=grid,
        in_specs=[
            pl.BlockSpec((blk, CONT_DIM), lambda i: (i, 0)),
            pl.BlockSpec((blk, PAD_DIM), lambda i: (i, 0)),
            pl.BlockSpec((blk, EMBED_DIM), lambda i: (i, 0)),
            pl.BlockSpec((CONT_DIM + 2 * EMBED_DIM, CONT_DIM),
                         lambda i: (0, 0)),
            pl.BlockSpec((PAD_DIM, CONT_DIM), lambda i: (0, 0)),
            pl.BlockSpec((1, CONT_DIM), lambda i: (0, 0)),
        ],
        out_specs=pl.BlockSpec((blk, CONT_DIM), lambda i: (i, 0)),
        out_shape=jax.ShapeDtypeStruct((BATCH, CONT_DIM), jnp.float32),
    )(x_cont, flag_emb_pad, class_emb, W, W1pad, b.reshape(1, CONT_DIM))


def kernel(x_cont, flag_idx, class_idx, flag_table, class_table, W, b):
    fidx = flag_idx.astype(jnp.int32)
    cidx = class_idx.astype(jnp.int32)
    flag_table_pad = jnp.pad(flag_table, ((0, 0), (0, PAD_DIM - EMBED_DIM)))
    W1pad = jnp.pad(W[CONT_DIM:CONT_DIM + EMBED_DIM, :],
                    ((0, PAD_DIM - EMBED_DIM), (0, 0)))
    flag_emb_pad, class_emb = _sc_gather(flag_table_pad, class_table,
                                         fidx, cidx)
    return _tc_matmul(x_cont, flag_emb_pad, class_emb, W, W1pad, b)


# speedup vs baseline: 1.6850x; 1.1662x over previous
import functools

import jax
import jax.numpy as jnp
from jax import lax
from jax.experimental import pallas as pl
from jax.experimental.pallas import tpu as pltpu
from jax.experimental.pallas import tpu_sc as plsc

NFLAGS = 100000
BATCH = 16384
EMBED_DIM = 32
CONT_DIM = 64

_NC = 2
_NS = 16
_NW = _NC * _NS
_B_PER_W = BATCH // _NW
_CHUNK = 128
_N_CHUNKS = _B_PER_W // _CHUNK

_VROWS = 102400
_Q = _VROWS // 4
_PREP_G = 1280
_PREP_STEPS = _Q // _PREP_G


def _prep_body(t0_ref, t1_ref, t2_ref, t3_ref, out_ref):
    out_ref[...] = jnp.concatenate(
        [t0_ref[...].T, t1_ref[...].T, t2_ref[...].T, t3_ref[...].T],
        axis=1)


def _prep(flag_table_t):
    nb = _Q // _PREP_G
    last = (NFLAGS - 1) // _PREP_G

    def _mk(j):
        return lambda i: (0, jnp.minimum(i + j * nb, last))

    return pl.pallas_call(
        _prep_body,
        grid=(_PREP_STEPS,),
        in_specs=[
            pl.BlockSpec((EMBED_DIM, _PREP_G), _mk(0)),
            pl.BlockSpec((EMBED_DIM, _PREP_G), _mk(1)),
            pl.BlockSpec((EMBED_DIM, _PREP_G), _mk(2)),
            pl.BlockSpec((EMBED_DIM, _PREP_G), _mk(3)),
        ],
        out_specs=pl.BlockSpec((_PREP_G, 4 * EMBED_DIM), lambda i: (i, 0)),
        out_shape=jax.ShapeDtypeStruct((_Q, 4 * EMBED_DIM), jnp.float32),
    )(flag_table_t, flag_table_t, flag_table_t, flag_table_t)


def _sc_gather_body(flag_tab, class_tab, fidx_hbm, cidx_hbm,
                    fout_hbm, cout_hbm,
                    fidx_v, cidx_v, frows_v, crows_v, fsem, csem):
    wid = lax.axis_index("s") * _NC + lax.axis_index("c")
    base = wid * _B_PER_W

    pltpu.sync_copy(fidx_hbm.at[pl.ds(base, _B_PER_W)], fidx_v)
    pltpu.sync_copy(cidx_hbm.at[pl.ds(base, _B_PER_W)], cidx_v)

    copies = []
    for j in range(_N_CHUNKS):
        copies.append(pltpu.async_copy(
            flag_tab.at[fidx_v.at[pl.ds(j * _CHUNK, _CHUNK)]],
            frows_v.at[pl.ds(j * _CHUNK, _CHUNK)], fsem))
        copies.append(pltpu.async_copy(
            class_tab.at[cidx_v.at[pl.ds(j * _CHUNK, _CHUNK)]],
            crows_v.at[pl.ds(j * _CHUNK, _CHUNK)], csem))
    for c in copies:
        c.wait()

    pltpu.sync_copy(frows_v, fout_hbm.at[pl.ds(base, _B_PER_W)])
    pltpu.sync_copy(crows_v, cout_hbm.at[pl.ds(base, _B_PER_W)])


def _sc_gather(flag_table_lin, class_table, fidx, cidx):
    mesh = plsc.VectorSubcoreMesh(core_axis_name="c", subcore_axis_name="s")
    kern = functools.partial(
        pl.kernel,
        mesh=mesh,
        out_type=[
            jax.ShapeDtypeStruct((BATCH, EMBED_DIM), jnp.float32),
            jax.ShapeDtypeStruct((BATCH, EMBED_DIM), jnp.float32),
        ],
        scratch_types=[
            pltpu.VMEM((_B_PER_W,), jnp.int32),
            pltpu.VMEM((_B_PER_W,), jnp.int32),
            pltpu.VMEM((_B_PER_W, EMBED_DIM), jnp.float32),
            pltpu.VMEM((_B_PER_W, EMBED_DIM), jnp.float32),
            pltpu.SemaphoreType.DMA,
            pltpu.SemaphoreType.DMA,
        ],
        compiler_params=pltpu.CompilerParams(use_tc_tiling_on_sc=False),
    )(_sc_gather_body)
    return kern(flag_table_lin, class_table, fidx, cidx)


def _tc_matmul_body(x_ref, f_ref, c_ref, w_ref, b_ref, o_ref):
    acc = jnp.dot(x_ref[...], w_ref[0:CONT_DIM, :],
                  preferred_element_type=jnp.float32)
    acc += jnp.dot(f_ref[...], w_ref[CONT_DIM:CONT_DIM + EMBED_DIM, :],
                   preferred_element_type=jnp.float32)
    acc += jnp.dot(c_ref[...], w_ref[CONT_DIM + EMBED_DIM:, :],
                   preferred_element_type=jnp.float32)
    o_ref[...] = acc + b_ref[...]


def _tc_matmul(x_cont, flag_emb, class_emb, W, b):
    blk = 2048
    grid = (BATCH // blk,)
    return pl.pallas_call(
        _tc_matmul_body,
        grid=grid,
        in_specs=[
            pl.BlockSpec((blk, CONT_DIM), lambda i: (i, 0)),
            pl.BlockSpec((blk, EMBED_DIM), lambda i: (i, 0)),
            pl.BlockSpec((blk, EMBED_DIM), lambda i: (i, 0)),
            pl.BlockSpec((CONT_DIM + 2 * EMBED_DIM, CONT_DIM),
                         lambda i: (0, 0)),
            pl.BlockSpec((1, CONT_DIM), lambda i: (0, 0)),
        ],
        out_specs=pl.BlockSpec((blk, CONT_DIM), lambda i: (i, 0)),
        out_shape=jax.ShapeDtypeStruct((BATCH, CONT_DIM), jnp.float32),
    )(x_cont, flag_emb, class_emb, W, b.reshape(1, CONT_DIM))


def kernel(x_cont, flag_idx, class_idx, flag_table, class_table, W, b):
    fidx = flag_idx.astype(jnp.int32)
    cidx = class_idx.astype(jnp.int32)
    fidx = 4 * (fidx % _Q) + fidx // _Q
    packed = _prep(flag_table.T)
    flag_table_lin = packed.reshape(_VROWS, EMBED_DIM)
    flag_emb, class_emb = _sc_gather(flag_table_lin, class_table, fidx, cidx)
    return _tc_matmul(x_cont, flag_emb, class_emb, W, b)


# speedup vs baseline: 1.9010x; 1.1282x over previous
import functools

import jax
import jax.numpy as jnp
from jax import lax
from jax.experimental import pallas as pl
from jax.experimental.pallas import tpu as pltpu
from jax.experimental.pallas import tpu_sc as plsc

NFLAGS = 100000
BATCH = 16384
EMBED_DIM = 32
CONT_DIM = 64

_NC = 2
_NS = 16
_NW = _NC * _NS
_B_PER_W = BATCH // _NW
_CHUNK = 128
_N_CHUNKS = _B_PER_W // _CHUNK

_VROWS = 102400
_Q = _VROWS // 4
_PREP_G = 1280
_PREP_STEPS = _Q // _PREP_G


def _prep_body(t0_ref, t1_ref, t2_ref, t3_ref, out_ref):
    cat = jnp.concatenate(
        [t0_ref[...], t1_ref[...], t2_ref[...], t3_ref[...]], axis=0)
    out_ref[...] = cat.T


def _prep(flag_table_t):
    nb = _Q // _PREP_G
    last = (NFLAGS - 1) // _PREP_G

    def _mk(j):
        return lambda i: (0, jnp.minimum(i + j * nb, last))

    return pl.pallas_call(
        _prep_body,
        grid=(_PREP_STEPS,),
        in_specs=[
            pl.BlockSpec((EMBED_DIM, _PREP_G), _mk(0)),
            pl.BlockSpec((EMBED_DIM, _PREP_G), _mk(1)),
            pl.BlockSpec((EMBED_DIM, _PREP_G), _mk(2)),
            pl.BlockSpec((EMBED_DIM, _PREP_G), _mk(3)),
        ],
        out_specs=pl.BlockSpec((_PREP_G, 4 * EMBED_DIM), lambda i: (i, 0)),
        out_shape=jax.ShapeDtypeStruct((_Q, 4 * EMBED_DIM), jnp.float32),
    )(flag_table_t, flag_table_t, flag_table_t, flag_table_t)


def _sc_gather_body(flag_tab, class_tab, fidx_hbm, cidx_hbm,
                    fout_hbm, cout_hbm,
                    fidx_v, cidx_v, frows_v, crows_v, fsem, csem):
    wid = lax.axis_index("s") * _NC + lax.axis_index("c")
    base = wid * _B_PER_W

    pltpu.sync_copy(fidx_hbm.at[pl.ds(base, _B_PER_W)], fidx_v)
    pltpu.sync_copy(cidx_hbm.at[pl.ds(base, _B_PER_W)], cidx_v)

    copies = []
    for j in range(_N_CHUNKS):
        copies.append(pltpu.async_copy(
            flag_tab.at[fidx_v.at[pl.ds(j * _CHUNK, _CHUNK)]],
            frows_v.at[pl.ds(j * _CHUNK, _CHUNK)], fsem))
        copies.append(pltpu.async_copy(
            class_tab.at[cidx_v.at[pl.ds(j * _CHUNK, _CHUNK)]],
            crows_v.at[pl.ds(j * _CHUNK, _CHUNK)], csem))
    for c in copies:
        c.wait()

    pltpu.sync_copy(frows_v, fout_hbm.at[pl.ds(base, _B_PER_W)])
    pltpu.sync_copy(crows_v, cout_hbm.at[pl.ds(base, _B_PER_W)])


def _sc_gather(flag_table_lin, class_table, fidx, cidx):
    mesh = plsc.VectorSubcoreMesh(core_axis_name="c", subcore_axis_name="s")
    kern = functools.partial(
        pl.kernel,
        mesh=mesh,
        out_type=[
            jax.ShapeDtypeStruct((BATCH, EMBED_DIM), jnp.float32),
            jax.ShapeDtypeStruct((BATCH, EMBED_DIM), jnp.float32),
        ],
        scratch_types=[
            pltpu.VMEM((_B_PER_W,), jnp.int32),
            pltpu.VMEM((_B_PER_W,), jnp.int32),
            pltpu.VMEM((_B_PER_W, EMBED_DIM), jnp.float32),
            pltpu.VMEM((_B_PER_W, EMBED_DIM), jnp.float32),
            pltpu.SemaphoreType.DMA,
            pltpu.SemaphoreType.DMA,
        ],
        compiler_params=pltpu.CompilerParams(use_tc_tiling_on_sc=False),
    )(_sc_gather_body)
    return kern(flag_table_lin, class_table, fidx, cidx)


def _tc_matmul_body(x_ref, f_ref, c_ref, w_ref, b_ref, o_ref):
    acc = jnp.dot(x_ref[...], w_ref[0:CONT_DIM, :],
                  preferred_element_type=jnp.float32)
    acc += jnp.dot(f_ref[...], w_ref[CONT_DIM:CONT_DIM + EMBED_DIM, :],
                   preferred_element_type=jnp.float32)
    acc += jnp.dot(c_ref[...], w_ref[CONT_DIM + EMBED_DIM:, :],
                   preferred_element_type=jnp.float32)
    o_ref[...] = acc + b_ref[...]


def _tc_matmul(x_cont, flag_emb, class_emb, W, b):
    blk = 2048
    grid = (BATCH // blk,)
    return pl.pallas_call(
        _tc_matmul_body,
        grid=grid,
        in_specs=[
            pl.BlockSpec((blk, CONT_DIM), lambda i: (i, 0)),
            pl.BlockSpec((blk, EMBED_DIM), lambda i: (i, 0)),
            pl.BlockSpec((blk, EMBED_DIM), lambda i: (i, 0)),
            pl.BlockSpec((CONT_DIM + 2 * EMBED_DIM, CONT_DIM),
                         lambda i: (0, 0)),
            pl.BlockSpec((1, CONT_DIM), lambda i: (0, 0)),
        ],
        out_specs=pl.BlockSpec((blk, CONT_DIM), lambda i: (i, 0)),
        out_shape=jax.ShapeDtypeStruct((BATCH, CONT_DIM), jnp.float32),
    )(x_cont, flag_emb, class_emb, W, b.reshape(1, CONT_DIM))


def kernel(x_cont, flag_idx, class_idx, flag_table, class_table, W, b):
    fidx = flag_idx.astype(jnp.int32)
    cidx = class_idx.astype(jnp.int32)
    fidx = 4 * (fidx % _Q) + fidx // _Q
    packed = _prep(flag_table.T)
    flag_table_lin = packed.reshape(_VROWS, EMBED_DIM)
    flag_emb, class_emb = _sc_gather(flag_table_lin, class_table, fidx, cidx)
    return _tc_matmul(x_cont, flag_emb, class_emb, W, b)


# speedup vs baseline: 1.9506x; 1.0261x over previous
import functools

import jax
import jax.numpy as jnp
from jax import lax
from jax.experimental import pallas as pl
from jax.experimental.pallas import tpu as pltpu
from jax.experimental.pallas import tpu_sc as plsc

NFLAGS = 100000
BATCH = 16384
EMBED_DIM = 32
CONT_DIM = 64

_NC = 2
_NS = 16
_NW = _NC * _NS
_B_PER_W = BATCH // _NW
_CHUNK = 128
_N_CHUNKS = _B_PER_W // _CHUNK

_VROWS = 102400
_Q = _VROWS // 4
_PREP_G = 1280
_PREP_STEPS = _Q // _PREP_G


def _prep_body(t0_ref, t1_ref, t2_ref, t3_ref, out_ref):
    cat = jnp.concatenate(
        [t0_ref[...], t1_ref[...], t2_ref[...], t3_ref[...]], axis=0)
    out_ref[...] = cat.T


def _prep(flag_table_t):
    nb = _Q // _PREP_G
    last = (NFLAGS - 1) // _PREP_G

    def _mk(j):
        return lambda i: (0, jnp.minimum(i + j * nb, last))

    return pl.pallas_call(
        _prep_body,
        grid=(_PREP_STEPS,),
        in_specs=[
            pl.BlockSpec((EMBED_DIM, _PREP_G), _mk(0)),
            pl.BlockSpec((EMBED_DIM, _PREP_G), _mk(1)),
            pl.BlockSpec((EMBED_DIM, _PREP_G), _mk(2)),
            pl.BlockSpec((EMBED_DIM, _PREP_G), _mk(3)),
        ],
        out_specs=pl.BlockSpec((_PREP_G, 4 * EMBED_DIM), lambda i: (i, 0)),
        out_shape=jax.ShapeDtypeStruct((_Q, 4 * EMBED_DIM), jnp.float32),
    )(flag_table_t, flag_table_t, flag_table_t, flag_table_t)


def _sc_gather_body(flag_tab, class_tab, fidx_hbm, cidx_hbm,
                    fout_hbm, cout_hbm,
                    fidx_v, cidx_v, frows_v, crows_v, fsem, csem):
    wid = lax.axis_index("s") * _NC + lax.axis_index("c")
    base = wid * _B_PER_W

    pltpu.sync_copy(fidx_hbm.at[pl.ds(base, _B_PER_W)], fidx_v)
    pltpu.sync_copy(cidx_hbm.at[pl.ds(base, _B_PER_W)], cidx_v)

    copies = []
    for j in range(_N_CHUNKS):
        copies.append(pltpu.async_copy(
            flag_tab.at[fidx_v.at[pl.ds(j * _CHUNK, _CHUNK)]],
            frows_v.at[pl.ds(j * _CHUNK, _CHUNK)], fsem))
        copies.append(pltpu.async_copy(
            class_tab.at[cidx_v.at[pl.ds(j * _CHUNK, _CHUNK)]],
            crows_v.at[pl.ds(j * _CHUNK, _CHUNK)], csem))
    for c in copies:
        c.wait()

    pltpu.sync_copy(frows_v, fout_hbm.at[wid])
    pltpu.sync_copy(crows_v, cout_hbm.at[wid])


def _sc_gather(flag_table_lin, class_table, fidx, cidx):
    mesh = plsc.VectorSubcoreMesh(core_axis_name="c", subcore_axis_name="s")
    kern = functools.partial(
        pl.kernel,
        mesh=mesh,
        out_type=[
            jax.ShapeDtypeStruct((_NW, _B_PER_W, EMBED_DIM), jnp.float32),
            jax.ShapeDtypeStruct((_NW, _B_PER_W, EMBED_DIM), jnp.float32),
        ],
        scratch_types=[
            pltpu.VMEM((_B_PER_W,), jnp.int32),
            pltpu.VMEM((_B_PER_W,), jnp.int32),
            pltpu.VMEM((_B_PER_W, EMBED_DIM), jnp.float32),
            pltpu.VMEM((_B_PER_W, EMBED_DIM), jnp.float32),
            pltpu.SemaphoreType.DMA,
            pltpu.SemaphoreType.DMA,
        ],
        compiler_params=pltpu.CompilerParams(use_tc_tiling_on_sc=False),
    )(_sc_gather_body)
    return kern(flag_table_lin, class_table, fidx, cidx)


_PK = 4
_PB = BATCH // _PK
_XW = _PK * CONT_DIM
_EW = _PK * EMBED_DIM


def _tc_matmul_body(x_ref, f_ref, c_ref, w0_ref, wf_ref, wc_ref, b_ref,
                    o_ref):
    acc = jnp.dot(x_ref[...], w0_ref[...],
                  preferred_element_type=jnp.float32)
    acc += jnp.dot(f_ref[...], wf_ref[...],
                   preferred_element_type=jnp.float32)
    acc += jnp.dot(c_ref[...], wc_ref[...],
                   preferred_element_type=jnp.float32)
    o_ref[...] = acc + b_ref[...]


def _tc_matmul(x_packed, f_packed, c_packed, Wbd0, WbdF, WbdC, b4):
    blk = 512
    grid = (_PB // blk,)
    return pl.pallas_call(
        _tc_matmul_body,
        grid=grid,
        in_specs=[
            pl.BlockSpec((blk, _XW), lambda i: (i, 0)),
            pl.BlockSpec((blk, _EW), lambda i: (i, 0)),
            pl.BlockSpec((blk, _EW), lambda i: (i, 0)),
            pl.BlockSpec((_XW, _XW), lambda i: (0, 0)),
            pl.BlockSpec((_EW, _XW), lambda i: (0, 0)),
            pl.BlockSpec((_EW, _XW), lambda i: (0, 0)),
            pl.BlockSpec((1, _XW), lambda i: (0, 0)),
        ],
        out_specs=pl.BlockSpec((blk, _XW), lambda i: (i, 0)),
        out_shape=jax.ShapeDtypeStruct((_PB, _XW), jnp.float32),
    )(x_packed, f_packed, c_packed, Wbd0, WbdF, WbdC, b4)


def kernel(x_cont, flag_idx, class_idx, flag_table, class_table, W, b):
    fidx = flag_idx.astype(jnp.int32)
    cidx = class_idx.astype(jnp.int32)
    fidx = 4 * (fidx % _Q) + fidx // _Q
    packed = _prep(flag_table.T)
    flag_table_lin = packed.reshape(_VROWS, EMBED_DIM)
    fout, cout = _sc_gather(flag_table_lin, class_table, fidx, cidx)

    eye4 = jnp.eye(_PK, dtype=jnp.float32)
    Wbd0 = jnp.kron(eye4, W[0:CONT_DIM, :])
    WbdF = jnp.kron(eye4, W[CONT_DIM:CONT_DIM + EMBED_DIM, :])
    WbdC = jnp.kron(eye4, W[CONT_DIM + EMBED_DIM:, :])
    b4 = jnp.tile(b, _PK).reshape(1, _XW)

    x_packed = x_cont.reshape(_PB, _XW)
    f_packed = fout.reshape(_PB, _EW)
    c_packed = cout.reshape(_PB, _EW)
    out_packed = _tc_matmul(x_packed, f_packed, c_packed,
                            Wbd0, WbdF, WbdC, b4)
    return out_packed.reshape(BATCH, CONT_DIM)


# speedup vs baseline: 2.3559x; 1.2078x over previous
import functools

import jax
import jax.numpy as jnp
from jax import lax
from jax.experimental import pallas as pl
from jax.experimental.pallas import tpu as pltpu
from jax.experimental.pallas import tpu_sc as plsc

NFLAGS = 100000
BATCH = 16384
EMBED_DIM = 32
CONT_DIM = 64

_NC = 2
_NS = 16
_NW = _NC * _NS
_B_PER_W = BATCH // _NW
_CHUNK = 128
_N_CHUNKS = _B_PER_W // _CHUNK

_VROWS = 102400
_Q = _VROWS // 4
_PREP_G = 1280
_PREP_STEPS = _Q // _PREP_G


def _prep_body(t0_ref, t1_ref, t2_ref, t3_ref, out_ref):
    cat = jnp.concatenate(
        [t0_ref[...], t1_ref[...], t2_ref[...], t3_ref[...]], axis=0)
    out_ref[...] = cat.T


def _prep(flag_table_t):
    nb = _Q // _PREP_G
    last = (NFLAGS - 1) // _PREP_G

    def _mk(j):
        return lambda i: (0, jnp.minimum(i + j * nb, last))

    return pl.pallas_call(
        _prep_body,
        grid=(_PREP_STEPS,),
        in_specs=[
            pl.BlockSpec((EMBED_DIM, _PREP_G), _mk(0)),
            pl.BlockSpec((EMBED_DIM, _PREP_G), _mk(1)),
            pl.BlockSpec((EMBED_DIM, _PREP_G), _mk(2)),
            pl.BlockSpec((EMBED_DIM, _PREP_G), _mk(3)),
        ],
        out_specs=pl.BlockSpec((_PREP_G, 4 * EMBED_DIM), lambda i: (i, 0)),
        out_shape=jax.ShapeDtypeStruct((_Q, 4 * EMBED_DIM), jnp.float32),
    )(flag_table_t, flag_table_t, flag_table_t, flag_table_t)


def _sc_gather_body(flag_tab, class_tab, fidx_hbm, cidx_hbm,
                    fout_hbm, cout_hbm,
                    fidx_v, cidx_v, frows_v, crows_v, fsem, csem):
    wid = lax.axis_index("s") * _NC + lax.axis_index("c")
    base = wid * _B_PER_W

    pltpu.sync_copy(fidx_hbm.at[pl.ds(base, _B_PER_W)], fidx_v)
    pltpu.sync_copy(cidx_hbm.at[pl.ds(base, _B_PER_W)], cidx_v)

    copies = []
    for j in range(_N_CHUNKS):
        copies.append(pltpu.async_copy(
            flag_tab.at[fidx_v.at[pl.ds(j * _CHUNK, _CHUNK)]],
            frows_v.at[pl.ds(j * _CHUNK, _CHUNK)], fsem))
        copies.append(pltpu.async_copy(
            class_tab.at[cidx_v.at[pl.ds(j * _CHUNK, _CHUNK)]],
            crows_v.at[pl.ds(j * _CHUNK, _CHUNK)], csem))
    for c in copies:
        c.wait()

    jq = wid // (_NW // _PK)
    g0 = (wid % (_NW // _PK)) * _B_PER_W
    pltpu.sync_copy(
        frows_v,
        fout_hbm.at[pl.ds(g0, _B_PER_W), pl.ds(jq * EMBED_DIM, EMBED_DIM)])
    pltpu.sync_copy(
        crows_v,
        cout_hbm.at[pl.ds(g0, _B_PER_W), pl.ds(jq * EMBED_DIM, EMBED_DIM)])


def _sc_gather(flag_table_lin, class_table, fidx, cidx):
    mesh = plsc.VectorSubcoreMesh(core_axis_name="c", subcore_axis_name="s")
    kern = functools.partial(
        pl.kernel,
        mesh=mesh,
        out_type=[
            jax.ShapeDtypeStruct((BATCH // 4, 4 * EMBED_DIM), jnp.float32),
            jax.ShapeDtypeStruct((BATCH // 4, 4 * EMBED_DIM), jnp.float32),
        ],
        scratch_types=[
            pltpu.VMEM((_B_PER_W,), jnp.int32),
            pltpu.VMEM((_B_PER_W,), jnp.int32),
            pltpu.VMEM((_B_PER_W, EMBED_DIM), jnp.float32),
            pltpu.VMEM((_B_PER_W, EMBED_DIM), jnp.float32),
            pltpu.SemaphoreType.DMA,
            pltpu.SemaphoreType.DMA,
        ],
        compiler_params=pltpu.CompilerParams(use_tc_tiling_on_sc=False),
    )(_sc_gather_body)
    return kern(flag_table_lin, class_table, fidx, cidx)


_PK = 4
_PB = BATCH // _PK
_XW = _PK * CONT_DIM
_EW = _PK * EMBED_DIM


_XG = 1024


def _xprep_body(t0_ref, t1_ref, t2_ref, t3_ref, out_ref):
    cat = jnp.concatenate(
        [t0_ref[...], t1_ref[...], t2_ref[...], t3_ref[...]], axis=0)
    out_ref[...] = cat.T


def _xprep(x_cont_t):
    nb = _PB // _XG

    def _mk(j):
        return lambda i: (0, i + j * nb)

    return pl.pallas_call(
        _xprep_body,
        grid=(_PB // _XG,),
        in_specs=[
            pl.BlockSpec((CONT_DIM, _XG), _mk(0)),
            pl.BlockSpec((CONT_DIM, _XG), _mk(1)),
            pl.BlockSpec((CONT_DIM, _XG), _mk(2)),
            pl.BlockSpec((CONT_DIM, _XG), _mk(3)),
        ],
        out_specs=pl.BlockSpec((_XG, _XW), lambda i: (i, 0)),
        out_shape=jax.ShapeDtypeStruct((_PB, _XW), jnp.float32),
    )(x_cont_t, x_cont_t, x_cont_t, x_cont_t)


def _tc_matmul_body(x_ref, f_ref, c_ref, w0_ref, wf_ref, wc_ref, b_ref,
                    o_ref):
    acc = jnp.dot(x_ref[...], w0_ref[...],
                  preferred_element_type=jnp.float32)
    acc += jnp.dot(f_ref[...], wf_ref[...],
                   preferred_element_type=jnp.float32)
    acc += jnp.dot(c_ref[...], wc_ref[...],
                   preferred_element_type=jnp.float32)
    o_ref[...] = (acc + b_ref[...]).T


def _tc_matmul(x_packed, f_packed, c_packed, Wbd0, WbdF, WbdC, b4):
    blk = 512
    grid = (_PB // blk,)
    return pl.pallas_call(
        _tc_matmul_body,
        grid=grid,
        in_specs=[
            pl.BlockSpec((blk, _XW), lambda i: (i, 0)),
            pl.BlockSpec((blk, _EW), lambda i: (i, 0)),
            pl.BlockSpec((blk, _EW), lambda i: (i, 0)),
            pl.BlockSpec((_XW, _XW), lambda i: (0, 0)),
            pl.BlockSpec((_EW, _XW), lambda i: (0, 0)),
            pl.BlockSpec((_EW, _XW), lambda i: (0, 0)),
            pl.BlockSpec((1, _XW), lambda i: (0, 0)),
        ],
        out_specs=pl.BlockSpec((_XW, blk), lambda i: (0, i)),
        out_shape=jax.ShapeDtypeStruct((_XW, _PB), jnp.float32),
    )(x_packed, f_packed, c_packed, Wbd0, WbdF, WbdC, b4)


def kernel(x_cont, flag_idx, class_idx, flag_table, class_table, W, b):
    fidx = flag_idx.astype(jnp.int32)
    cidx = class_idx.astype(jnp.int32)
    fidx = 4 * (fidx % _Q) + fidx // _Q
    packed = _prep(flag_table.T)
    flag_table_lin = packed.reshape(_VROWS, EMBED_DIM)
    f_packed, c_packed = _sc_gather(flag_table_lin, class_table, fidx, cidx)

    eye4 = jnp.eye(_PK, dtype=jnp.float32)
    perm = jnp.arange(_XW)
    perm = (perm % _PK) * CONT_DIM + perm // _PK
    Wbd0 = jnp.kron(eye4, W[0:CONT_DIM, :])[:, perm]
    WbdF = jnp.kron(eye4, W[CONT_DIM:CONT_DIM + EMBED_DIM, :])[:, perm]
    WbdC = jnp.kron(eye4, W[CONT_DIM + EMBED_DIM:, :])[:, perm]
    b4 = jnp.repeat(b, _PK).reshape(1, _XW)

    x_packed = _xprep(x_cont.T)
    out_t = _tc_matmul(x_packed, f_packed, c_packed, Wbd0, WbdF, WbdC, b4)
    return out_t.reshape(CONT_DIM, BATCH).T
